# baseline (speedup 1.0000x reference)
import functools

import jax
import jax.numpy as jnp
from jax.experimental import pallas as pl

N_BLK = 2000


def _layer_body(agg_ref, cnt_ref, x_ref, wl_ref, wr_ref, b_ref, o_ref, *, mode):
    cnt = jnp.maximum(cnt_ref[...], 1.0)
    mean = agg_ref[...] / cnt
    h = (jnp.dot(mean, wl_ref[...], preferred_element_type=jnp.float32)
         + jnp.dot(x_ref[...], wr_ref[...], preferred_element_type=jnp.float32)
         + b_ref[...][None, :])
    if mode == "relu":
        h = jnp.maximum(h, 0.0)
    elif mode == "log_softmax":
        m = jnp.max(h, axis=1, keepdims=True)
        h = h - (m + jnp.log(jnp.sum(jnp.exp(h - m), axis=1, keepdims=True)))
    o_ref[...] = h


def _dense_layer(agg, cnt, x, wl, wr, b, mode):
    n, d_in = x.shape
    d_out = wl.shape[1]
    grid = n // N_BLK
    return pl.pallas_call(
        functools.partial(_layer_body, mode=mode),
        grid=(grid,),
        in_specs=[
            pl.BlockSpec((N_BLK, agg.shape[1]), lambda i: (i, 0)),
            pl.BlockSpec((N_BLK, 1), lambda i: (i, 0)),
            pl.BlockSpec((N_BLK, d_in), lambda i: (i, 0)),
            pl.BlockSpec(wl.shape, lambda i: (0, 0)),
            pl.BlockSpec(wr.shape, lambda i: (0, 0)),
            pl.BlockSpec(b.shape, lambda i: (0,)),
        ],
        out_specs=pl.BlockSpec((N_BLK, d_out), lambda i: (i, 0)),
        out_shape=jax.ShapeDtypeStruct((n, d_out), jnp.float32),
    )(agg, cnt, x, wl, wr, b)


def kernel(x, edge_index, W1l, W1r, b1, W2l, W2r, b2, W3l, W3r, b3):
    n = x.shape[0]
    src = edge_index[0].astype(jnp.int32)
    dst = edge_index[1].astype(jnp.int32)
    cnt = jax.ops.segment_sum(jnp.ones_like(dst, jnp.float32), dst,
                              num_segments=n)[:, None]

    agg1 = jax.ops.segment_sum(jnp.take(x, src, axis=0), dst, num_segments=n)
    h1 = _dense_layer(agg1, cnt, x, W1l, W1r, b1, "relu")
    agg2 = jax.ops.segment_sum(jnp.take(h1, src, axis=0), dst, num_segments=n)
    h2 = _dense_layer(agg2, cnt, h1, W2l, W2r, b2, "relu")
    agg3 = jax.ops.segment_sum(jnp.take(h2, src, axis=0), dst, num_segments=n)
    return _dense_layer(agg3, cnt, h2, W3l, W3r, b3, "log_softmax")


# speedup vs baseline: 15.0254x; 15.0254x over previous
import functools

import jax
import jax.numpy as jnp
from jax import lax
from jax.experimental import pallas as pl
from jax.experimental.pallas import tpu as pltpu
from jax.experimental.pallas import tpu_sc as plsc

N = 100000
D = 16
PAD_SEG = 2400
PADN = N + PAD_SEG
ROWS_PER_TILE = PADN // 16
ZROWS = ROWS_PER_TILE // 8
LANE = 128
K = 16
EPAD_ROWS = 25088
ROWS_PER_CORE = EPAD_ROWS // 2
BLOCKS_PER_CORE = ROWS_PER_CORE // K
ITERS_PER_TILE = BLOCKS_PER_CORE // 16
N_BLK = 400

_MESH = plsc.VectorSubcoreMesh(core_axis_name="c", subcore_axis_name="s")
_F32 = jnp.float32


def _edge_loop(c, s, src_hbm, dst_hbm, sidx, didx, val, bufs, gsems, ssems,
               acc, sync_scat_fn):
    rowbase0 = c * ROWS_PER_CORE
    b0, b1, b2, b3 = bufs
    g0s, g1s, g2s, g3s = gsems

    def ag(jrow, buf, sem):
        return pltpu.async_copy(val.at[sidx.at[jrow]], buf, sem)

    @pl.loop(0, ITERS_PER_TILE)
    def _(i):
        rb = rowbase0 + (s + i * 16) * K
        pltpu.sync_copy(src_hbm.at[pl.ds(rb, K)], sidx)
        pltpu.sync_copy(dst_hbm.at[pl.ds(rb, K)], didx)

        if sync_scat_fn is not None:
            @pl.loop(0, K // 4)
            def _(h):
                jb = h * 4
                h0 = ag(jb, b0, g0s)
                h1 = ag(jb + 1, b1, g1s)
                h0.wait()
                sync_scat_fn(b0, didx.at[jb])
                h2 = ag(jb + 2, b0, g0s)
                h1.wait()
                sync_scat_fn(b1, didx.at[jb + 1])
                h3 = ag(jb + 3, b1, g1s)
                h2.wait()
                sync_scat_fn(b0, didx.at[jb + 2])
                h3.wait()
                sync_scat_fn(b1, didx.at[jb + 3])
        else:
            s0s, s1s, s2s, s3s = ssems

            def as_(buf, jrow, sem):
                return pltpu.async_copy(buf, acc.at[didx.at[jrow]], sem,
                                        add=True)

            @pl.loop(0, K // 8)
            def _(h):
                jb = h * 8
                g0 = ag(jb, b0, g0s)
                g1 = ag(jb + 1, b1, g1s)
                g2 = ag(jb + 2, b2, g2s)
                g0.wait()
                s0 = as_(b0, jb, s0s)
                g3 = ag(jb + 3, b3, g3s)
                g1.wait()
                s1 = as_(b1, jb + 1, s1s)
                s0.wait()
                g4 = ag(jb + 4, b0, g0s)
                g2.wait()
                s2 = as_(b2, jb + 2, s2s)
                s1.wait()
                g5 = ag(jb + 5, b1, g1s)
                g3.wait()
                s3 = as_(b3, jb + 3, s3s)
                s2.wait()
                g6 = ag(jb + 6, b2, g2s)
                g4.wait()
                s4 = as_(b0, jb + 4, s0s)
                s3.wait()
                g7 = ag(jb + 7, b3, g3s)
                g5.wait()
                s5 = as_(b1, jb + 5, s1s)
                g6.wait()
                s6 = as_(b2, jb + 6, s2s)
                g7.wait()
                s7 = as_(b3, jb + 7, s3s)
                s4.wait()
                s5.wait()
                s6.wait()
                s7.wait()


def _agg_body(nv, with_cnt, *refs):
    src_hbm, dst_hbm = refs[0], refs[1]
    k = 2
    vals = refs[k:k + nv]; k += nv
    z2d = refs[k]; k += 1
    if with_cnt:
        z1d, ones1 = refs[k:k + 2]; k += 2
    outs = refs[k:k + nv]; k += nv
    if with_cnt:
        cnt_out = refs[k]; k += 1
    sidx, didx, rowsa, rowsb, rowsc, rowsd = refs[k:k + 6]; k += 6
    if with_cnt:
        ones_v = refs[k]; k += 1
    acc = refs[k]; k += 1
    if with_cnt:
        cntacc = refs[k]; k += 1
    gsems = refs[k:k + 4]; k += 4
    ssems = refs[k:k + 4]

    c = lax.axis_index("c")
    s = lax.axis_index("s")
    nbase = s * ROWS_PER_TILE
    off = c * PADN + nbase

    if with_cnt:
        pltpu.sync_copy(ones1, ones_v)

    for p in range(nv):
        pltpu.sync_copy(z2d, acc.at[pl.ds(nbase, ROWS_PER_TILE)])
        cnt_now = with_cnt and p == 0
        if cnt_now:
            pltpu.sync_copy(z1d, cntacc.at[pl.ds(nbase, ROWS_PER_TILE)])
        plsc.subcore_barrier()

        if cnt_now:
            def scat(buf, drow):
                pltpu.sync_copy(buf, acc.at[drow], add=True)
                pltpu.sync_copy(ones_v, cntacc.at[drow], add=True)
        else:
            scat = None

        _edge_loop(c, s, src_hbm, dst_hbm, sidx, didx, vals[p],
                   (rowsa, rowsb, rowsc, rowsd), gsems, ssems, acc, scat)
        plsc.subcore_barrier()
        for t in range(8):
            pltpu.sync_copy(acc.at[pl.ds(nbase + t * ZROWS, ZROWS)],
                            outs[p].at[pl.ds(off + t * ZROWS, ZROWS)])
        if cnt_now:
            for t in range(5):
                pltpu.sync_copy(cntacc.at[pl.ds(nbase + t * 1280, 1280)],
                                cnt_out.at[pl.ds(off + t * 1280, 1280)])


def _sc_agg(srcM, dstM, vals, z2d, cnt_aux=None):
    nv = len(vals)
    with_cnt = cnt_aux is not None
    out_type = [jax.ShapeDtypeStruct((2 * PADN, D), _F32)] * nv
    if with_cnt:
        out_type = out_type + [jax.ShapeDtypeStruct((2 * PADN,), _F32)]
    scratch = [
        pltpu.VMEM((K, LANE), jnp.int32),
        pltpu.VMEM((K, LANE), jnp.int32),
        pltpu.VMEM((LANE, D), _F32),
        pltpu.VMEM((LANE, D), _F32),
        pltpu.VMEM((LANE, D), _F32),
        pltpu.VMEM((LANE, D), _F32),
    ]
    if with_cnt:
        scratch += [pltpu.VMEM((LANE,), _F32)]
    scratch += [pltpu.VMEM_SHARED((PADN, D), _F32)]
    if with_cnt:
        scratch += [pltpu.VMEM_SHARED((PADN,), _F32)]
    scratch += [pltpu.SemaphoreType.DMA] * 8
    f = pl.kernel(
        functools.partial(_agg_body, nv, with_cnt),
        out_type=out_type,
        mesh=_MESH,
        scratch_types=scratch,
        compiler_params=pltpu.CompilerParams(use_tc_tiling_on_sc=False),
    )
    args = (srcM, dstM, *vals, z2d) + (tuple(cnt_aux) if with_cnt else ())
    out = f(*args)
    return list(out) if isinstance(out, (list, tuple)) else [out]


def _split(part):
    return part[:N], part[PADN:PADN + N]


def _mean_cat(parts, cnt):
    return jnp.concatenate([(p[...] + q[...]) / cnt for p, q in parts], axis=1)


def _tc1_body(a00, a01, c0, c1, x, wl, wr, b, *outs):
    cnt = jnp.maximum(c0[...] + c1[...], 1.0)
    mean = (a00[...] + a01[...]) / cnt
    h = (jnp.dot(mean, wl[...], preferred_element_type=_F32)
         + jnp.dot(x[...], wr[...], preferred_element_type=_F32)
         + b[...][None, :])
    h = jnp.maximum(h, 0.0)
    for j, o in enumerate(outs):
        o[...] = h[:, D * j:D * (j + 1)]


def _tc2_body(*refs):
    aggs = refs[:8]
    c0, c1 = refs[8:10]
    h1t = refs[10:14]
    w2l, w2r, b2, w3l, w3r = refs[14:19]
    p2o, s2o = refs[19:21]
    cnt = jnp.maximum(c0[...] + c1[...], 1.0)
    mean = _mean_cat(list(zip(aggs[0::2], aggs[1::2])), cnt)
    h1 = jnp.concatenate([t[...] for t in h1t], axis=1)
    h2 = (jnp.dot(mean, w2l[...], preferred_element_type=_F32)
          + jnp.dot(h1, w2r[...], preferred_element_type=_F32)
          + b2[...][None, :])
    h2 = jnp.maximum(h2, 0.0)
    p2 = jnp.dot(h2, w3l[...], preferred_element_type=_F32)
    p2o[...] = jnp.concatenate(
        [p2, jnp.zeros((p2.shape[0], D - 2), _F32)], axis=1)
    s2o[...] = jnp.dot(h2, w3r[...], preferred_element_type=_F32)


def _tc3_body(a0, a1, c0, c1, s2, b3, o):
    cnt = jnp.maximum(c0[...] + c1[...], 1.0)
    z = (a0[...] + a1[...])[:, :2] / cnt + s2[...] + b3[...][None, :]
    m = jnp.max(z, axis=1, keepdims=True)
    o[...] = z - (m + jnp.log(jnp.sum(jnp.exp(z - m), axis=1, keepdims=True)))


def _row_spec(d):
    return pl.BlockSpec((N_BLK, d), lambda i: (i, 0))


def _full_spec(shape):
    nd = len(shape)
    return pl.BlockSpec(shape, (lambda i: (0,) * nd))


def _tc1(a00, a01, c0, c1, x, wl, wr, b):
    return pl.pallas_call(
        _tc1_body,
        grid=(N // N_BLK,),
        in_specs=[_row_spec(D)] * 2 + [_row_spec(1)] * 2 + [_row_spec(16)]
                 + [_full_spec(wl.shape), _full_spec(wr.shape),
                    _full_spec(b.shape)],
        out_specs=[_row_spec(D)] * 4,
        out_shape=[jax.ShapeDtypeStruct((N, D), _F32)] * 4,
    )(a00, a01, c0, c1, x, wl, wr, b)


def _tc2(aggs, c0, c1, h1t, w2l, w2r, b2, w3l, w3r):
    return pl.pallas_call(
        _tc2_body,
        grid=(N // N_BLK,),
        in_specs=[_row_spec(D)] * 8 + [_row_spec(1)] * 2
                 + [_row_spec(D)] * 4
                 + [_full_spec(w2l.shape), _full_spec(w2r.shape),
                    _full_spec(b2.shape), _full_spec(w3l.shape),
                    _full_spec(w3r.shape)],
        out_specs=[_row_spec(D), _row_spec(2)],
        out_shape=[jax.ShapeDtypeStruct((N, D), _F32),
                   jax.ShapeDtypeStruct((N, 2), _F32)],
    )(*aggs, c0, c1, *h1t, w2l, w2r, b2, w3l, w3r)


def _tc3(a0, a1, c0, c1, s2, b3):
    return pl.pallas_call(
        _tc3_body,
        grid=(N // N_BLK,),
        in_specs=[_row_spec(D), _row_spec(D), _row_spec(1), _row_spec(1),
                  _row_spec(2), _full_spec(b3.shape)],
        out_specs=_row_spec(2),
        out_shape=jax.ShapeDtypeStruct((N, 2), _F32),
    )(a0, a1, c0, c1, s2, b3)


def kernel(x, edge_index, W1l, W1r, b1, W2l, W2r, b2, W3l, W3r, b3):
    e = edge_index.shape[1]
    src = edge_index[0].astype(jnp.int32)
    dst = edge_index[1].astype(jnp.int32)
    npad = EPAD_ROWS * LANE - e
    pad_i = jnp.arange(npad, dtype=jnp.int32)
    srcM = jnp.concatenate([src, pad_i % N]).reshape(EPAD_ROWS, LANE)
    dstM = jnp.concatenate([dst, N + pad_i % PAD_SEG]).reshape(EPAD_ROWS, LANE)
    z2d = jnp.zeros((ROWS_PER_TILE, D), _F32)
    z1d = jnp.zeros((ROWS_PER_TILE,), _F32)
    ones1 = jnp.ones((LANE,), _F32)

    out1 = _sc_agg(srcM, dstM, [x], z2d, cnt_aux=(z1d, ones1))
    a00, a01 = _split(out1[0])
    cnt0, cnt1 = _split(out1[1])
    c0, c1 = cnt0[:, None], cnt1[:, None]

    h1t = _tc1(a00, a01, c0, c1, x, W1l, W1r, b1)

    agg2 = _sc_agg(srcM, dstM, list(h1t), z2d)
    aggs2 = []
    for p in agg2:
        aggs2.extend(_split(p))

    p2pad, s2 = _tc2(aggs2, c0, c1, h1t, W2l, W2r, b2, W3l, W3r)

    agg3 = _sc_agg(srcM, dstM, [p2pad], z2d)[0]
    a30, a31 = _split(agg3)
    return _tc3(a30, a31, c0, c1, s2, b3)


# speedup vs baseline: 17.8474x; 1.1878x over previous
import functools

import jax
import jax.numpy as jnp
from jax import lax
from jax.experimental import pallas as pl
from jax.experimental.pallas import tpu as pltpu
from jax.experimental.pallas import tpu_sc as plsc

N = 100000
D = 16
PAD_SEG = 2400
PADN = N + PAD_SEG
ROWS_PER_TILE = PADN // 16
ZROWS = ROWS_PER_TILE // 8
LANE = 128
K = 16
EPAD_ROWS = 25088
ROWS_PER_CORE = EPAD_ROWS // 2
BLOCKS_PER_CORE = ROWS_PER_CORE // K
ITERS_PER_TILE = BLOCKS_PER_CORE // 16
N_BLK = 800
OFF_BLKS = PADN // N_BLK

_MESH = plsc.VectorSubcoreMesh(core_axis_name="c", subcore_axis_name="s")
_F32 = jnp.float32


def _edge_loop(c, s, src_hbm, dst_hbm, sidx, didx, val, bufs, gsems, ssems,
               acc, sync_scat_fn):
    rowbase0 = c * ROWS_PER_CORE
    b0, b1, b2, b3 = bufs
    g0s, g1s, g2s, g3s = gsems

    def ag(jrow, buf, sem):
        return pltpu.async_copy(val.at[sidx.at[jrow]], buf, sem)

    @pl.loop(0, ITERS_PER_TILE)
    def _(i):
        rb = rowbase0 + (s + i * 16) * K
        pltpu.sync_copy(src_hbm.at[pl.ds(rb, K)], sidx)
        pltpu.sync_copy(dst_hbm.at[pl.ds(rb, K)], didx)

        if sync_scat_fn is not None:
            @pl.loop(0, K // 4)
            def _(h):
                jb = h * 4
                h0 = ag(jb, b0, g0s)
                h1 = ag(jb + 1, b1, g1s)
                h0.wait()
                sync_scat_fn(b0, didx.at[jb])
                h2 = ag(jb + 2, b0, g0s)
                h1.wait()
                sync_scat_fn(b1, didx.at[jb + 1])
                h3 = ag(jb + 3, b1, g1s)
                h2.wait()
                sync_scat_fn(b0, didx.at[jb + 2])
                h3.wait()
                sync_scat_fn(b1, didx.at[jb + 3])
        else:
            s0s, s1s, s2s, s3s = ssems

            def as_(buf, jrow, sem):
                return pltpu.async_copy(buf, acc.at[didx.at[jrow]], sem,
                                        add=True)

            @pl.loop(0, K // 8)
            def _(h):
                jb = h * 8
                g0 = ag(jb, b0, g0s)
                g1 = ag(jb + 1, b1, g1s)
                g2 = ag(jb + 2, b2, g2s)
                g0.wait()
                s0 = as_(b0, jb, s0s)
                g3 = ag(jb + 3, b3, g3s)
                g1.wait()
                s1 = as_(b1, jb + 1, s1s)
                s0.wait()
                g4 = ag(jb + 4, b0, g0s)
                g2.wait()
                s2 = as_(b2, jb + 2, s2s)
                s1.wait()
                g5 = ag(jb + 5, b1, g1s)
                g3.wait()
                s3 = as_(b3, jb + 3, s3s)
                s2.wait()
                g6 = ag(jb + 6, b2, g2s)
                g4.wait()
                s4 = as_(b0, jb + 4, s0s)
                s3.wait()
                g7 = ag(jb + 7, b3, g3s)
                g5.wait()
                s5 = as_(b1, jb + 5, s1s)
                g6.wait()
                s6 = as_(b2, jb + 6, s2s)
                g7.wait()
                s7 = as_(b3, jb + 7, s3s)
                s4.wait()
                s5.wait()
                s6.wait()
                s7.wait()


def _agg_body(nv, with_cnt, *refs):
    src_hbm, dst_hbm = refs[0], refs[1]
    k = 2
    vals = refs[k:k + nv]; k += nv
    z2d = refs[k]; k += 1
    if with_cnt:
        z1d, ones1 = refs[k:k + 2]; k += 2
    outs = refs[k:k + nv]; k += nv
    if with_cnt:
        cnt_out = refs[k]; k += 1
    sidx, didx, rowsa, rowsb, rowsc, rowsd = refs[k:k + 6]; k += 6
    if with_cnt:
        ones_v = refs[k]; k += 1
    acc = refs[k]; k += 1
    if with_cnt:
        cntacc = refs[k]; k += 1
    gsems = refs[k:k + 4]; k += 4
    ssems = refs[k:k + 4]

    c = lax.axis_index("c")
    s = lax.axis_index("s")
    nbase = s * ROWS_PER_TILE
    off = c * PADN + nbase

    if with_cnt:
        pltpu.sync_copy(ones1, ones_v)

    for p in range(nv):
        pltpu.sync_copy(z2d, acc.at[pl.ds(nbase, ROWS_PER_TILE)])
        cnt_now = with_cnt and p == 0
        if cnt_now:
            pltpu.sync_copy(z1d, cntacc.at[pl.ds(nbase, ROWS_PER_TILE)])
        plsc.subcore_barrier()

        if cnt_now:
            def scat(buf, drow):
                pltpu.sync_copy(buf, acc.at[drow], add=True)
                pltpu.sync_copy(ones_v, cntacc.at[drow], add=True)
        else:
            scat = None

        _edge_loop(c, s, src_hbm, dst_hbm, sidx, didx, vals[p],
                   (rowsa, rowsb, rowsc, rowsd), gsems, ssems, acc, scat)
        plsc.subcore_barrier()
        for t in range(8):
            pltpu.sync_copy(acc.at[pl.ds(nbase + t * ZROWS, ZROWS)],
                            outs[p].at[pl.ds(off + t * ZROWS, ZROWS)])
        if cnt_now:
            for t in range(5):
                pltpu.sync_copy(cntacc.at[pl.ds(nbase + t * 1280, 1280)],
                                cnt_out.at[pl.ds(off + t * 1280, 1280)])


def _sc_agg(srcM, dstM, vals, z2d, cnt_aux=None):
    nv = len(vals)
    with_cnt = cnt_aux is not None
    out_type = [jax.ShapeDtypeStruct((2 * PADN, D), _F32)] * nv
    if with_cnt:
        out_type = out_type + [jax.ShapeDtypeStruct((2 * PADN,), _F32)]
    scratch = [
        pltpu.VMEM((K, LANE), jnp.int32),
        pltpu.VMEM((K, LANE), jnp.int32),
        pltpu.VMEM((LANE, D), _F32),
        pltpu.VMEM((LANE, D), _F32),
        pltpu.VMEM((LANE, D), _F32),
        pltpu.VMEM((LANE, D), _F32),
    ]
    if with_cnt:
        scratch += [pltpu.VMEM((LANE,), _F32)]
    scratch += [pltpu.VMEM_SHARED((PADN, D), _F32)]
    if with_cnt:
        scratch += [pltpu.VMEM_SHARED((PADN,), _F32)]
    scratch += [pltpu.SemaphoreType.DMA] * 8
    f = pl.kernel(
        functools.partial(_agg_body, nv, with_cnt),
        out_type=out_type,
        mesh=_MESH,
        scratch_types=scratch,
        compiler_params=pltpu.CompilerParams(use_tc_tiling_on_sc=False),
    )
    args = (srcM, dstM, *vals, z2d) + (tuple(cnt_aux) if with_cnt else ())
    out = f(*args)
    return list(out) if isinstance(out, (list, tuple)) else [out]


def _split(part):
    return part[:N], part[PADN:PADN + N]


def _mean_cat(parts, cnt):
    return jnp.concatenate([(p[...] + q[...]) / cnt for p, q in parts], axis=1)


def _tc1_body(a00, a01, c0, c1, x, wl, wr, b, *outs):
    cnt = jnp.maximum(c0[...] + c1[...], 1.0)
    mean = (a00[...] + a01[...]) / cnt
    h = (jnp.dot(mean, wl[...], preferred_element_type=_F32)
         + jnp.dot(x[...], wr[...], preferred_element_type=_F32)
         + b[...][None, :])
    h = jnp.maximum(h, 0.0)
    for j, o in enumerate(outs):
        o[...] = h[:, D * j:D * (j + 1)]


def _tc2_body(*refs):
    aggs = refs[:8]
    c0, c1 = refs[8:10]
    h1t = refs[10:14]
    w2l, w2r, b2, w3l, w3r = refs[14:19]
    p2o, s2o = refs[19:21]
    cnt = jnp.maximum(c0[...] + c1[...], 1.0)
    mean = _mean_cat(list(zip(aggs[0::2], aggs[1::2])), cnt)
    h1 = jnp.concatenate([t[...] for t in h1t], axis=1)
    h2 = (jnp.dot(mean, w2l[...], preferred_element_type=_F32)
          + jnp.dot(h1, w2r[...], preferred_element_type=_F32)
          + b2[...][None, :])
    h2 = jnp.maximum(h2, 0.0)
    p2 = jnp.dot(h2, w3l[...], preferred_element_type=_F32)
    p2o[...] = jnp.concatenate(
        [p2, jnp.zeros((p2.shape[0], D - 2), _F32)], axis=1)
    s2o[...] = jnp.dot(h2, w3r[...], preferred_element_type=_F32)


def _tc3_body(a0, a1, c0, c1, s2, b3, o):
    cnt = jnp.maximum(c0[...] + c1[...], 1.0)
    z = (a0[...] + a1[...])[:, :2] / cnt + s2[...] + b3[...][None, :]
    m = jnp.max(z, axis=1, keepdims=True)
    o[...] = z - (m + jnp.log(jnp.sum(jnp.exp(z - m), axis=1, keepdims=True)))


def _row_spec(d):
    return pl.BlockSpec((N_BLK, d), lambda i: (i, 0))


def _part_spec(d, part):
    return pl.BlockSpec((N_BLK, d), lambda i, _p=part: (i + _p * OFF_BLKS, 0))


def _full_spec(shape):
    nd = len(shape)
    return pl.BlockSpec(shape, (lambda i: (0,) * nd))


def _tc1(aggp, cntp, x, wl, wr, b):
    return pl.pallas_call(
        _tc1_body,
        grid=(N // N_BLK,),
        in_specs=[_part_spec(D, 0), _part_spec(D, 1),
                  _part_spec(1, 0), _part_spec(1, 1), _row_spec(16)]
                 + [_full_spec(wl.shape), _full_spec(wr.shape),
                    _full_spec(b.shape)],
        out_specs=[_row_spec(D)] * 4,
        out_shape=[jax.ShapeDtypeStruct((N, D), _F32)] * 4,
    )(aggp, aggp, cntp, cntp, x, wl, wr, b)


def _tc2(agg2, cntp, h1t, w2l, w2r, b2, w3l, w3r):
    aggs = []
    specs = []
    for a in agg2:
        aggs += [a, a]
        specs += [_part_spec(D, 0), _part_spec(D, 1)]
    return pl.pallas_call(
        _tc2_body,
        grid=(N // N_BLK,),
        in_specs=specs + [_part_spec(1, 0), _part_spec(1, 1)]
                 + [_row_spec(D)] * 4
                 + [_full_spec(w2l.shape), _full_spec(w2r.shape),
                    _full_spec(b2.shape), _full_spec(w3l.shape),
                    _full_spec(w3r.shape)],
        out_specs=[_row_spec(D), _row_spec(2)],
        out_shape=[jax.ShapeDtypeStruct((N, D), _F32),
                   jax.ShapeDtypeStruct((N, 2), _F32)],
    )(*aggs, cntp, cntp, *h1t, w2l, w2r, b2, w3l, w3r)


def _tc3(agg3, cntp, s2, b3):
    return pl.pallas_call(
        _tc3_body,
        grid=(N // N_BLK,),
        in_specs=[_part_spec(D, 0), _part_spec(D, 1),
                  _part_spec(1, 0), _part_spec(1, 1),
                  _row_spec(2), _full_spec(b3.shape)],
        out_specs=_row_spec(2),
        out_shape=jax.ShapeDtypeStruct((N, 2), _F32),
    )(agg3, agg3, cntp, cntp, s2, b3)


def kernel(x, edge_index, W1l, W1r, b1, W2l, W2r, b2, W3l, W3r, b3):
    e = edge_index.shape[1]
    src = edge_index[0].astype(jnp.int32)
    dst = edge_index[1].astype(jnp.int32)
    npad = EPAD_ROWS * LANE - e
    pad_i = jnp.arange(npad, dtype=jnp.int32)
    srcM = jnp.concatenate([src, pad_i % N]).reshape(EPAD_ROWS, LANE)
    dstM = jnp.concatenate([dst, N + pad_i % PAD_SEG]).reshape(EPAD_ROWS, LANE)
    z2d = jnp.zeros((ROWS_PER_TILE, D), _F32)
    z1d = jnp.zeros((ROWS_PER_TILE,), _F32)
    ones1 = jnp.ones((LANE,), _F32)

    out1 = _sc_agg(srcM, dstM, [x], z2d, cnt_aux=(z1d, ones1))
    cntp = out1[1][:, None]

    h1t = _tc1(out1[0], cntp, x, W1l, W1r, b1)

    agg2 = _sc_agg(srcM, dstM, list(h1t), z2d)

    p2pad, s2 = _tc2(agg2, cntp, h1t, W2l, W2r, b2, W3l, W3r)

    agg3 = _sc_agg(srcM, dstM, [p2pad], z2d)[0]
    return _tc3(agg3, cntp, s2, b3)


# speedup vs baseline: 18.6460x; 1.0447x over previous
import functools

import jax
import jax.numpy as jnp
from jax import lax
from jax.experimental import pallas as pl
from jax.experimental.pallas import tpu as pltpu
from jax.experimental.pallas import tpu_sc as plsc

N = 100000
D = 16
PAD_SEG = 2400
PADN = N + PAD_SEG
ROWS_PER_TILE = PADN // 16
ZROWS = ROWS_PER_TILE // 8
LANE = 128
K = 16
EPAD_ROWS = 25088
ROWS_PER_CORE = EPAD_ROWS // 2
BLOCKS_PER_CORE = ROWS_PER_CORE // K
ITERS_PER_TILE = BLOCKS_PER_CORE // 16
N_BLK = 800
OFF_BLKS = PADN // N_BLK

_MESH = plsc.VectorSubcoreMesh(core_axis_name="c", subcore_axis_name="s")
_F32 = jnp.float32


def _edge_loop(c, s, src_hbm, dst_hbm, sidx, didx, val, bufs, gsems, ssems,
               acc, cnt_refs):
    rowbase0 = c * ROWS_PER_CORE
    b0, b1, b2, b3 = bufs
    g0s, g1s, g2s, g3s = gsems

    def ag(jrow, buf, sem):
        return pltpu.async_copy(val.at[sidx.at[jrow]], buf, sem)

    @pl.loop(0, ITERS_PER_TILE)
    def _(i):
        rb = rowbase0 + (s + i * 16) * K
        pltpu.sync_copy(src_hbm.at[pl.ds(rb, K)], sidx)
        pltpu.sync_copy(dst_hbm.at[pl.ds(rb, K)], didx)

        s0s, s1s, s2s, s3s = ssems

        def as_(buf, jrow, sem):
            return pltpu.async_copy(buf, acc.at[didx.at[jrow]], sem,
                                    add=True)

        if cnt_refs is not None:
            ones_v, cntacc = cnt_refs

            @pl.loop(0, K // 4)
            def _(h):
                jb = h * 4
                g0 = ag(jb, b0, g0s)
                g1 = ag(jb + 1, b1, g1s)
                g2 = ag(jb + 2, b2, g2s)
                g3 = ag(jb + 3, b3, g3s)
                g0.wait()
                s0 = as_(b0, jb, s0s)
                pltpu.sync_copy(ones_v, cntacc.at[didx.at[jb]], add=True)
                g1.wait()
                s1 = as_(b1, jb + 1, s1s)
                pltpu.sync_copy(ones_v, cntacc.at[didx.at[jb + 1]], add=True)
                g2.wait()
                s2 = as_(b2, jb + 2, s2s)
                pltpu.sync_copy(ones_v, cntacc.at[didx.at[jb + 2]], add=True)
                g3.wait()
                s3 = as_(b3, jb + 3, s3s)
                pltpu.sync_copy(ones_v, cntacc.at[didx.at[jb + 3]], add=True)
                s0.wait()
                s1.wait()
                s2.wait()
                s3.wait()
        else:

            @pl.loop(0, K // 8)
            def _(h):
                jb = h * 8
                g0 = ag(jb, b0, g0s)
                g1 = ag(jb + 1, b1, g1s)
                g2 = ag(jb + 2, b2, g2s)
                g0.wait()
                s0 = as_(b0, jb, s0s)
                g3 = ag(jb + 3, b3, g3s)
                g1.wait()
                s1 = as_(b1, jb + 1, s1s)
                s0.wait()
                g4 = ag(jb + 4, b0, g0s)
                g2.wait()
                s2 = as_(b2, jb + 2, s2s)
                s1.wait()
                g5 = ag(jb + 5, b1, g1s)
                g3.wait()
                s3 = as_(b3, jb + 3, s3s)
                s2.wait()
                g6 = ag(jb + 6, b2, g2s)
                g4.wait()
                s4 = as_(b0, jb + 4, s0s)
                s3.wait()
                g7 = ag(jb + 7, b3, g3s)
                g5.wait()
                s5 = as_(b1, jb + 5, s1s)
                g6.wait()
                s6 = as_(b2, jb + 6, s2s)
                g7.wait()
                s7 = as_(b3, jb + 7, s3s)
                s4.wait()
                s5.wait()
                s6.wait()
                s7.wait()


def _agg_body(nv, with_cnt, *refs):
    src_hbm, dst_hbm = refs[0], refs[1]
    k = 2
    vals = refs[k:k + nv]; k += nv
    z2d = refs[k]; k += 1
    if with_cnt:
        z1d, ones1 = refs[k:k + 2]; k += 2
    outs = refs[k:k + nv]; k += nv
    if with_cnt:
        cnt_out = refs[k]; k += 1
    sidx, didx, rowsa, rowsb, rowsc, rowsd = refs[k:k + 6]; k += 6
    if with_cnt:
        ones_v = refs[k]; k += 1
    acc = refs[k]; k += 1
    if with_cnt:
        cntacc = refs[k]; k += 1
    gsems = refs[k:k + 4]; k += 4
    ssems = refs[k:k + 4]

    c = lax.axis_index("c")
    s = lax.axis_index("s")
    nbase = s * ROWS_PER_TILE
    off = c * PADN + nbase

    if with_cnt:
        pltpu.sync_copy(ones1, ones_v)

    for p in range(nv):
        pltpu.sync_copy(z2d, acc.at[pl.ds(nbase, ROWS_PER_TILE)])
        cnt_now = with_cnt and p == 0
        if cnt_now:
            pltpu.sync_copy(z1d, cntacc.at[pl.ds(nbase, ROWS_PER_TILE)])
        plsc.subcore_barrier()

        cr = (ones_v, cntacc) if cnt_now else None
        _edge_loop(c, s, src_hbm, dst_hbm, sidx, didx, vals[p],
                   (rowsa, rowsb, rowsc, rowsd), gsems, ssems, acc, cr)
        plsc.subcore_barrier()
        for t in range(8):
            pltpu.sync_copy(acc.at[pl.ds(nbase + t * ZROWS, ZROWS)],
                            outs[p].at[pl.ds(off + t * ZROWS, ZROWS)])
        if cnt_now:
            for t in range(5):
                pltpu.sync_copy(cntacc.at[pl.ds(nbase + t * 1280, 1280)],
                                cnt_out.at[pl.ds(off + t * 1280, 1280)])


def _sc_agg(srcM, dstM, vals, z2d, cnt_aux=None):
    nv = len(vals)
    with_cnt = cnt_aux is not None
    out_type = [jax.ShapeDtypeStruct((2 * PADN, D), _F32)] * nv
    if with_cnt:
        out_type = out_type + [jax.ShapeDtypeStruct((2 * PADN,), _F32)]
    scratch = [
        pltpu.VMEM((K, LANE), jnp.int32),
        pltpu.VMEM((K, LANE), jnp.int32),
        pltpu.VMEM((LANE, D), _F32),
        pltpu.VMEM((LANE, D), _F32),
        pltpu.VMEM((LANE, D), _F32),
        pltpu.VMEM((LANE, D), _F32),
    ]
    if with_cnt:
        scratch += [pltpu.VMEM((LANE,), _F32)]
    scratch += [pltpu.VMEM_SHARED((PADN, D), _F32)]
    if with_cnt:
        scratch += [pltpu.VMEM_SHARED((PADN,), _F32)]
    scratch += [pltpu.SemaphoreType.DMA] * 8
    f = pl.kernel(
        functools.partial(_agg_body, nv, with_cnt),
        out_type=out_type,
        mesh=_MESH,
        scratch_types=scratch,
        compiler_params=pltpu.CompilerParams(use_tc_tiling_on_sc=False),
    )
    args = (srcM, dstM, *vals, z2d) + (tuple(cnt_aux) if with_cnt else ())
    out = f(*args)
    return list(out) if isinstance(out, (list, tuple)) else [out]


def _split(part):
    return part[:N], part[PADN:PADN + N]


def _mean_cat(parts, cnt):
    return jnp.concatenate([(p[...] + q[...]) / cnt for p, q in parts], axis=1)


def _tc1_body(a00, a01, c0, c1, x, wl, wr, b, *outs):
    cnt = jnp.maximum(c0[...] + c1[...], 1.0)
    mean = (a00[...] + a01[...]) / cnt
    h = (jnp.dot(mean, wl[...], preferred_element_type=_F32)
         + jnp.dot(x[...], wr[...], preferred_element_type=_F32)
         + b[...][None, :])
    h = jnp.maximum(h, 0.0)
    for j, o in enumerate(outs):
        o[...] = h[:, D * j:D * (j + 1)]


def _tc2_body(*refs):
    aggs = refs[:8]
    c0, c1 = refs[8:10]
    h1t = refs[10:14]
    w2l, w2r, b2, w3l, w3r = refs[14:19]
    p2o, s2o = refs[19:21]
    cnt = jnp.maximum(c0[...] + c1[...], 1.0)
    mean = _mean_cat(list(zip(aggs[0::2], aggs[1::2])), cnt)
    h1 = jnp.concatenate([t[...] for t in h1t], axis=1)
    h2 = (jnp.dot(mean, w2l[...], preferred_element_type=_F32)
          + jnp.dot(h1, w2r[...], preferred_element_type=_F32)
          + b2[...][None, :])
    h2 = jnp.maximum(h2, 0.0)
    p2 = jnp.dot(h2, w3l[...], preferred_element_type=_F32)
    p2o[...] = jnp.concatenate(
        [p2, jnp.zeros((p2.shape[0], D - 2), _F32)], axis=1)
    s2o[...] = jnp.dot(h2, w3r[...], preferred_element_type=_F32)


def _tc3_body(a0, a1, c0, c1, s2, b3, o):
    cnt = jnp.maximum(c0[...] + c1[...], 1.0)
    z = (a0[...] + a1[...])[:, :2] / cnt + s2[...] + b3[...][None, :]
    m = jnp.max(z, axis=1, keepdims=True)
    o[...] = z - (m + jnp.log(jnp.sum(jnp.exp(z - m), axis=1, keepdims=True)))


def _row_spec(d):
    return pl.BlockSpec((N_BLK, d), lambda i: (i, 0))


def _part_spec(d, part):
    return pl.BlockSpec((N_BLK, d), lambda i, _p=part: (i + _p * OFF_BLKS, 0))


def _full_spec(shape):
    nd = len(shape)
    return pl.BlockSpec(shape, (lambda i: (0,) * nd))


def _tc1(aggp, cntp, x, wl, wr, b):
    return pl.pallas_call(
        _tc1_body,
        grid=(N // N_BLK,),
        in_specs=[_part_spec(D, 0), _part_spec(D, 1),
                  _part_spec(1, 0), _part_spec(1, 1), _row_spec(16)]
                 + [_full_spec(wl.shape), _full_spec(wr.shape),
                    _full_spec(b.shape)],
        out_specs=[_row_spec(D)] * 4,
        out_shape=[jax.ShapeDtypeStruct((N, D), _F32)] * 4,
    )(aggp, aggp, cntp, cntp, x, wl, wr, b)


def _tc2(agg2, cntp, h1t, w2l, w2r, b2, w3l, w3r):
    aggs = []
    specs = []
    for a in agg2:
        aggs += [a, a]
        specs += [_part_spec(D, 0), _part_spec(D, 1)]
    return pl.pallas_call(
        _tc2_body,
        grid=(N // N_BLK,),
        in_specs=specs + [_part_spec(1, 0), _part_spec(1, 1)]
                 + [_row_spec(D)] * 4
                 + [_full_spec(w2l.shape), _full_spec(w2r.shape),
                    _full_spec(b2.shape), _full_spec(w3l.shape),
                    _full_spec(w3r.shape)],
        out_specs=[_row_spec(D), _row_spec(2)],
        out_shape=[jax.ShapeDtypeStruct((N, D), _F32),
                   jax.ShapeDtypeStruct((N, 2), _F32)],
    )(*aggs, cntp, cntp, *h1t, w2l, w2r, b2, w3l, w3r)


def _tc3(agg3, cntp, s2, b3):
    return pl.pallas_call(
        _tc3_body,
        grid=(N // N_BLK,),
        in_specs=[_part_spec(D, 0), _part_spec(D, 1),
                  _part_spec(1, 0), _part_spec(1, 1),
                  _row_spec(2), _full_spec(b3.shape)],
        out_specs=_row_spec(2),
        out_shape=jax.ShapeDtypeStruct((N, 2), _F32),
    )(agg3, agg3, cntp, cntp, s2, b3)


def kernel(x, edge_index, W1l, W1r, b1, W2l, W2r, b2, W3l, W3r, b3):
    e = edge_index.shape[1]
    src = edge_index[0].astype(jnp.int32)
    dst = edge_index[1].astype(jnp.int32)
    npad = EPAD_ROWS * LANE - e
    pad_i = jnp.arange(npad, dtype=jnp.int32)
    srcM = jnp.concatenate([src, pad_i % N]).reshape(EPAD_ROWS, LANE)
    dstM = jnp.concatenate([dst, N + pad_i % PAD_SEG]).reshape(EPAD_ROWS, LANE)
    z2d = jnp.zeros((ROWS_PER_TILE, D), _F32)
    z1d = jnp.zeros((ROWS_PER_TILE,), _F32)
    ones1 = jnp.ones((LANE,), _F32)

    out1 = _sc_agg(srcM, dstM, [x], z2d, cnt_aux=(z1d, ones1))
    cntp = out1[1][:, None]

    h1t = _tc1(out1[0], cntp, x, W1l, W1r, b1)

    agg2 = _sc_agg(srcM, dstM, list(h1t), z2d)

    p2pad, s2 = _tc2(agg2, cntp, h1t, W2l, W2r, b2, W3l, W3r)

    agg3 = _sc_agg(srcM, dstM, [p2pad], z2d)[0]
    return _tc3(agg3, cntp, s2, b3)
